# baseline (speedup 1.0000x reference)
import jax
import jax.numpy as jnp
from jax import lax
from jax.experimental import pallas as pl
from jax.experimental.pallas import tpu as pltpu
from jax.experimental.pallas import tpu_sc as plsc

N_NODES = 99999
N_EDGES = N_NODES * 64
N_OUT = N_NODES // 3
ACC = 33792
X_PAD = 100000

NC, NS = 2, 16
NW = NC * NS
BLK = 4096
NBLK = 49
W_E = NBLK * BLK
E_PAD = NW * W_E
ZSEG = ACC // NS


def _sc_scatter(x_hbm, src_hbm, dst_hbm, p_hbm,
                x_v, src_v, dst_v, idx_v, val_v, seg_v, acc_sh, sem):
    cid = lax.axis_index("c")
    sid = lax.axis_index("s")
    wid = sid * NC + cid

    if True:
        pltpu.sync_copy(x_hbm, x_v)

        def zstep(j, c):
            seg_v[pl.ds(j * 16, 16)] = jnp.zeros((16,), jnp.float32)
            return c
        lax.fori_loop(0, ZSEG // 16, zstep, 0)
        pltpu.sync_copy(seg_v, acc_sh.at[pl.ds(sid * ZSEG, ZSEG)])
        plsc.subcore_barrier()

        base = wid * W_E
        three = jnp.int32(3)

        def block(g, c):
            boff = base + g * BLK
            pltpu.sync_copy(src_hbm.at[pl.ds(boff, BLK)], src_v)
            pltpu.sync_copy(dst_hbm.at[pl.ds(boff, BLK)], dst_v)

            def chunk(t, c2):
                co = t * 1024
                handles = []
                for k in range(8):
                    for j in range(8):
                        o = co + k * 128 + j * 16
                        s16 = src_v[pl.ds(o, 16)]
                        d16 = dst_v[pl.ds(o, 16)]
                        xs = plsc.load_gather(x_v, [s16])
                        q = lax.div(d16, three)
                        r = d16 - q * three
                        idx_v[k, pl.ds(j * 16, 16)] = q
                        val_v[k, pl.ds(j * 16, 16)] = jnp.where(
                            r == 0, xs, jnp.float32(0.0))
                    handles.append(pltpu.async_copy(
                        val_v.at[k], acc_sh.at[idx_v.at[k]], sem, add=True))
                for h in handles:
                    h.wait()
                return c2
            lax.fori_loop(0, BLK // 1024, chunk, 0)
            return c
        lax.fori_loop(0, NBLK, block, 0)
        plsc.subcore_barrier()

        pltpu.sync_copy(acc_sh.at[pl.ds(sid * ZSEG, ZSEG)], seg_v)
        pltpu.sync_copy(seg_v, p_hbm.at[pl.ds(cid * ACC + sid * ZSEG, ZSEG)])


def _combine(consts_ref, p0_ref, p1_ref, x3_ref, out_ref):
    wl = consts_ref[0]
    wr = consts_ref[1]
    w10 = consts_ref[2]
    w11 = consts_ref[3]
    b10 = consts_ref[4]
    b11 = consts_ref[5]
    w20 = consts_ref[6]
    w21 = consts_ref[7]
    b2v = consts_ref[8]
    h = wl * (p0_ref[...] + p1_ref[...]) + wr * x3_ref[...]
    a0 = jnp.maximum(w10 * h + b10, 0.0)
    a1 = jnp.maximum(w11 * h + b11, 0.0)
    out_ref[...] = w20 * a0 + w21 * a1 + b2v


def kernel(x, edge_index, W_l, W_r, w1, b1, w2, b2):
    xf = x.reshape(-1)
    x_p = jnp.concatenate(
        [xf, jnp.zeros((X_PAD - N_NODES,), jnp.float32)])
    src = edge_index[0]
    dst = edge_index[1]
    npad = E_PAD - N_EDGES
    src_p = jnp.concatenate([src, jnp.zeros((npad,), jnp.int32)])
    dst_p = jnp.concatenate([dst, jnp.ones((npad,), jnp.int32)])

    mesh = plsc.VectorSubcoreMesh(core_axis_name="c", subcore_axis_name="s")
    partials = pl.kernel(
        _sc_scatter,
        out_type=jax.ShapeDtypeStruct((NC * ACC,), jnp.float32),
        mesh=mesh,
        compiler_params=pltpu.CompilerParams(needs_layout_passes=False),
        scratch_types=[
            pltpu.VMEM((X_PAD,), jnp.float32),
            pltpu.VMEM((BLK,), jnp.int32),
            pltpu.VMEM((BLK,), jnp.int32),
            pltpu.VMEM((8, 128), jnp.int32),
            pltpu.VMEM((8, 128), jnp.float32),
            pltpu.VMEM((ZSEG,), jnp.float32),
            pltpu.VMEM_SHARED((ACC,), jnp.float32),
            pltpu.SemaphoreType.DMA,
        ],
    )(x_p, src_p, dst_p)

    x3 = xf[: N_OUT * 3].reshape(N_OUT, 3)[:, 0]
    x3_p = jnp.concatenate(
        [x3, jnp.zeros((ACC - N_OUT,), jnp.float32)]).reshape(264, 128)
    consts = jnp.concatenate([
        W_l.reshape(-1), W_r.reshape(-1), w1.reshape(-1),
        b1.reshape(-1), w2.reshape(-1), b2.reshape(-1),
        jnp.zeros((7,), jnp.float32),
    ])

    out2d = pl.pallas_call(
        _combine,
        out_shape=jax.ShapeDtypeStruct((264, 128), jnp.float32),
        in_specs=[
            pl.BlockSpec(memory_space=pltpu.SMEM),
            pl.BlockSpec(memory_space=pltpu.VMEM),
            pl.BlockSpec(memory_space=pltpu.VMEM),
            pl.BlockSpec(memory_space=pltpu.VMEM),
        ],
        out_specs=pl.BlockSpec(memory_space=pltpu.VMEM),
    )(consts, partials[:ACC].reshape(264, 128),
      partials[ACC:].reshape(264, 128), x3_p)

    return out2d.reshape(-1)[:N_OUT]


# speedup vs baseline: 1.7947x; 1.7947x over previous
import jax
import jax.numpy as jnp
from jax import lax
from jax.experimental import pallas as pl
from jax.experimental.pallas import tpu as pltpu
from jax.experimental.pallas import tpu_sc as plsc

N_NODES = 99999
N_EDGES = N_NODES * 64
N_OUT = N_NODES // 3
ACC = 33792
X_PAD = 100000

NC, NS = 2, 16
NW = NC * NS
BLK = 4096
NBLK = 49
W_E = NBLK * BLK
E_PAD = NW * W_E
ZSEG = ACC // NS


def _sc_scatter(x_hbm, src_hbm, dst_hbm, p_hbm,
                x_v, src_v, dst_v, idx_v, val_v, seg_v, acc_sh, sem):
    cid = lax.axis_index("c")
    sid = lax.axis_index("s")
    wid = sid * NC + cid

    if True:
        pltpu.sync_copy(x_hbm, x_v)

        def zstep(j, c):
            seg_v[pl.ds(j * 16, 16)] = jnp.zeros((16,), jnp.float32)
            return c
        lax.fori_loop(0, ZSEG // 16, zstep, 0)
        pltpu.sync_copy(seg_v, acc_sh.at[pl.ds(sid * ZSEG, ZSEG)])
        plsc.subcore_barrier()

        base = wid * W_E
        three = jnp.int32(3)

        def block(g, c):
            boff = base + g * BLK
            pltpu.sync_copy(src_hbm.at[pl.ds(boff, BLK)], src_v)
            pltpu.sync_copy(dst_hbm.at[pl.ds(boff, BLK)], dst_v)

            def group(t, c2):
                co = t * 128
                for j in range(8):
                    o = co + j * 16
                    s16 = src_v[pl.ds(o, 16)]
                    d16 = dst_v[pl.ds(o, 16)]
                    xs = plsc.load_gather(x_v, [s16])
                    q = lax.div(d16, three)
                    r = d16 - q * three
                    idx_v[pl.ds(j * 16, 16)] = q
                    val_v[pl.ds(j * 16, 16)] = jnp.where(
                        r == 0, xs, jnp.float32(0.0))
                pltpu.sync_copy(val_v, acc_sh.at[idx_v], add=True)
                return c2
            lax.fori_loop(0, BLK // 128, group, 0)
            return c
        lax.fori_loop(0, NBLK, block, 0)
        plsc.subcore_barrier()

        pltpu.sync_copy(acc_sh.at[pl.ds(sid * ZSEG, ZSEG)], seg_v)
        pltpu.sync_copy(seg_v, p_hbm.at[pl.ds(cid * ACC + sid * ZSEG, ZSEG)])


def _combine(consts_ref, p0_ref, p1_ref, x3_ref, out_ref):
    wl = consts_ref[0]
    wr = consts_ref[1]
    w10 = consts_ref[2]
    w11 = consts_ref[3]
    b10 = consts_ref[4]
    b11 = consts_ref[5]
    w20 = consts_ref[6]
    w21 = consts_ref[7]
    b2v = consts_ref[8]
    h = wl * (p0_ref[...] + p1_ref[...]) + wr * x3_ref[...]
    a0 = jnp.maximum(w10 * h + b10, 0.0)
    a1 = jnp.maximum(w11 * h + b11, 0.0)
    out_ref[...] = w20 * a0 + w21 * a1 + b2v


def kernel(x, edge_index, W_l, W_r, w1, b1, w2, b2):
    xf = x.reshape(-1)
    x_p = jnp.concatenate(
        [xf, jnp.zeros((X_PAD - N_NODES,), jnp.float32)])
    src = edge_index[0]
    dst = edge_index[1]
    npad = E_PAD - N_EDGES
    src_p = jnp.concatenate([src, jnp.zeros((npad,), jnp.int32)])
    dst_p = jnp.concatenate([dst, jnp.ones((npad,), jnp.int32)])

    mesh = plsc.VectorSubcoreMesh(core_axis_name="c", subcore_axis_name="s")
    partials = pl.kernel(
        _sc_scatter,
        out_type=jax.ShapeDtypeStruct((NC * ACC,), jnp.float32),
        mesh=mesh,
        compiler_params=pltpu.CompilerParams(needs_layout_passes=False),
        scratch_types=[
            pltpu.VMEM((X_PAD,), jnp.float32),
            pltpu.VMEM((BLK,), jnp.int32),
            pltpu.VMEM((BLK,), jnp.int32),
            pltpu.VMEM((128,), jnp.int32),
            pltpu.VMEM((128,), jnp.float32),
            pltpu.VMEM((ZSEG,), jnp.float32),
            pltpu.VMEM_SHARED((ACC,), jnp.float32),
            pltpu.SemaphoreType.DMA,
        ],
    )(x_p, src_p, dst_p)

    x3 = xf[: N_OUT * 3].reshape(N_OUT, 3)[:, 0]
    x3_p = jnp.concatenate(
        [x3, jnp.zeros((ACC - N_OUT,), jnp.float32)]).reshape(264, 128)
    consts = jnp.concatenate([
        W_l.reshape(-1), W_r.reshape(-1), w1.reshape(-1),
        b1.reshape(-1), w2.reshape(-1), b2.reshape(-1),
        jnp.zeros((7,), jnp.float32),
    ])

    out2d = pl.pallas_call(
        _combine,
        out_shape=jax.ShapeDtypeStruct((264, 128), jnp.float32),
        in_specs=[
            pl.BlockSpec(memory_space=pltpu.SMEM),
            pl.BlockSpec(memory_space=pltpu.VMEM),
            pl.BlockSpec(memory_space=pltpu.VMEM),
            pl.BlockSpec(memory_space=pltpu.VMEM),
        ],
        out_specs=pl.BlockSpec(memory_space=pltpu.VMEM),
    )(consts, partials[:ACC].reshape(264, 128),
      partials[ACC:].reshape(264, 128), x3_p)

    return out2d.reshape(-1)[:N_OUT]


# speedup vs baseline: 2.2501x; 1.2538x over previous
import jax
import jax.numpy as jnp
from jax import lax
from jax.experimental import pallas as pl
from jax.experimental.pallas import tpu as pltpu
from jax.experimental.pallas import tpu_sc as plsc

N_NODES = 99999
N_EDGES = N_NODES * 64
N_OUT = N_NODES // 3
ACC = 33792
X_PAD = 100000

NC, NS = 2, 16
NW = NC * NS
BLK = 4096
NBLK = 49
W_E = NBLK * BLK
E_PAD = NW * W_E
ZSEG = ACC // NS


def _sc_scatter(x_hbm, src_hbm, dst_hbm, p_hbm,
                x_v, src_v, dst_v, idx_v, val_v, seg_v, acc_sh, sem):
    cid = lax.axis_index("c")
    sid = lax.axis_index("s")
    wid = sid * NC + cid

    if True:
        pltpu.sync_copy(x_hbm, x_v)

        def zstep(j, c):
            seg_v[pl.ds(j * 16, 16)] = jnp.zeros((16,), jnp.float32)
            return c
        lax.fori_loop(0, ZSEG // 16, zstep, 0)
        pltpu.sync_copy(seg_v, acc_sh.at[pl.ds(sid * ZSEG, ZSEG)])
        plsc.subcore_barrier()

        base = wid * W_E
        three = jnp.int32(3)

        def block(g, c):
            boff = base + g * BLK
            pltpu.sync_copy(src_hbm.at[pl.ds(boff, BLK)], src_v)
            pltpu.sync_copy(dst_hbm.at[pl.ds(boff, BLK)], dst_v)

            def group(t, c2):
                co = t * 128
                for j in range(8):
                    o = co + j * 16
                    s16 = src_v[pl.ds(o, 16)]
                    d16 = dst_v[pl.ds(o, 16)]
                    xs = plsc.load_gather(x_v, [s16])
                    q = lax.div(d16, three)
                    r = d16 - q * three
                    idx_v[pl.ds(co + j * 16, 16)] = q
                    val_v[pl.ds(co + j * 16, 16)] = jnp.where(
                        r == 0, xs, jnp.float32(0.0))
                return c2
            lax.fori_loop(0, BLK // 128, group, 0)
            pltpu.sync_copy(val_v, acc_sh.at[idx_v], add=True)
            return c
        lax.fori_loop(0, NBLK, block, 0)
        plsc.subcore_barrier()

        pltpu.sync_copy(acc_sh.at[pl.ds(sid * ZSEG, ZSEG)], seg_v)
        pltpu.sync_copy(seg_v, p_hbm.at[pl.ds(cid * ACC + sid * ZSEG, ZSEG)])


def _combine(consts_ref, p0_ref, p1_ref, x3_ref, out_ref):
    wl = consts_ref[0]
    wr = consts_ref[1]
    w10 = consts_ref[2]
    w11 = consts_ref[3]
    b10 = consts_ref[4]
    b11 = consts_ref[5]
    w20 = consts_ref[6]
    w21 = consts_ref[7]
    b2v = consts_ref[8]
    h = wl * (p0_ref[...] + p1_ref[...]) + wr * x3_ref[...]
    a0 = jnp.maximum(w10 * h + b10, 0.0)
    a1 = jnp.maximum(w11 * h + b11, 0.0)
    out_ref[...] = w20 * a0 + w21 * a1 + b2v


def kernel(x, edge_index, W_l, W_r, w1, b1, w2, b2):
    xf = x.reshape(-1)
    x_p = jnp.concatenate(
        [xf, jnp.zeros((X_PAD - N_NODES,), jnp.float32)])
    src = edge_index[0]
    dst = edge_index[1]
    npad = E_PAD - N_EDGES
    src_p = jnp.concatenate([src, jnp.zeros((npad,), jnp.int32)])
    dst_p = jnp.concatenate([dst, jnp.ones((npad,), jnp.int32)])

    mesh = plsc.VectorSubcoreMesh(core_axis_name="c", subcore_axis_name="s")
    partials = pl.kernel(
        _sc_scatter,
        out_type=jax.ShapeDtypeStruct((NC * ACC,), jnp.float32),
        mesh=mesh,
        compiler_params=pltpu.CompilerParams(needs_layout_passes=False),
        scratch_types=[
            pltpu.VMEM((X_PAD,), jnp.float32),
            pltpu.VMEM((BLK,), jnp.int32),
            pltpu.VMEM((BLK,), jnp.int32),
            pltpu.VMEM((BLK,), jnp.int32),
            pltpu.VMEM((BLK,), jnp.float32),
            pltpu.VMEM((ZSEG,), jnp.float32),
            pltpu.VMEM_SHARED((ACC,), jnp.float32),
            pltpu.SemaphoreType.DMA,
        ],
    )(x_p, src_p, dst_p)

    x3 = xf[: N_OUT * 3].reshape(N_OUT, 3)[:, 0]
    x3_p = jnp.concatenate(
        [x3, jnp.zeros((ACC - N_OUT,), jnp.float32)]).reshape(264, 128)
    consts = jnp.concatenate([
        W_l.reshape(-1), W_r.reshape(-1), w1.reshape(-1),
        b1.reshape(-1), w2.reshape(-1), b2.reshape(-1),
        jnp.zeros((7,), jnp.float32),
    ])

    out2d = pl.pallas_call(
        _combine,
        out_shape=jax.ShapeDtypeStruct((264, 128), jnp.float32),
        in_specs=[
            pl.BlockSpec(memory_space=pltpu.SMEM),
            pl.BlockSpec(memory_space=pltpu.VMEM),
            pl.BlockSpec(memory_space=pltpu.VMEM),
            pl.BlockSpec(memory_space=pltpu.VMEM),
        ],
        out_specs=pl.BlockSpec(memory_space=pltpu.VMEM),
    )(consts, partials[:ACC].reshape(264, 128),
      partials[ACC:].reshape(264, 128), x3_p)

    return out2d.reshape(-1)[:N_OUT]


# speedup vs baseline: 2.4550x; 1.0911x over previous
import jax
import jax.numpy as jnp
from jax import lax
from jax.experimental import pallas as pl
from jax.experimental.pallas import tpu as pltpu
from jax.experimental.pallas import tpu_sc as plsc

N_NODES = 99999
N_EDGES = N_NODES * 64
N_OUT = N_NODES // 3
ACC = 33792
X_PAD = 100000

NC, NS = 2, 16
NW = NC * NS
BLK = 4096
NBLK = 50
W_E = NBLK * BLK
E_PAD = NW * W_E
ZSEG = ACC // NS


def _sc_scatter(x_hbm, src_hbm, dst_hbm, p_hbm,
                x_v, src_v, dst_v, idx_v, val_v, idx_w, val_w,
                seg_v, acc_sh, sem, sem2):
    cid = lax.axis_index("c")
    sid = lax.axis_index("s")
    wid = sid * NC + cid

    if True:
        pltpu.sync_copy(x_hbm, x_v)

        def zstep(j, c):
            seg_v[pl.ds(j * 16, 16)] = jnp.zeros((16,), jnp.float32)
            return c
        lax.fori_loop(0, ZSEG // 16, zstep, 0)
        pltpu.sync_copy(seg_v, acc_sh.at[pl.ds(sid * ZSEG, ZSEG)])
        plsc.subcore_barrier()

        base = wid * W_E
        three = jnp.int32(3)

        def fill(boff, idx_b, val_b):
            pltpu.sync_copy(src_hbm.at[pl.ds(boff, BLK)], src_v)
            pltpu.sync_copy(dst_hbm.at[pl.ds(boff, BLK)], dst_v)

            def group(t, c2):
                co = t * 128
                for j in range(8):
                    o = co + j * 16
                    s16 = src_v[pl.ds(o, 16)]
                    d16 = dst_v[pl.ds(o, 16)]
                    xs = plsc.load_gather(x_v, [s16])
                    q = lax.div(d16, three)
                    r = d16 - q * three
                    idx_b[pl.ds(o, 16)] = q
                    val_b[pl.ds(o, 16)] = jnp.where(
                        r == 0, xs, jnp.float32(0.0))
                return c2
            lax.fori_loop(0, BLK // 128, group, 0)

        def pair(h, c):
            boff = base + h * (2 * BLK)

            @pl.when(h > 0)
            def _():
                pltpu.make_async_copy(
                    val_v, acc_sh.at[idx_v], sem).wait()
            fill(boff, idx_v, val_v)
            pltpu.async_copy(val_v, acc_sh.at[idx_v], sem, add=True)

            @pl.when(h > 0)
            def _():
                pltpu.make_async_copy(
                    val_w, acc_sh.at[idx_w], sem2).wait()
            fill(boff + BLK, idx_w, val_w)
            pltpu.async_copy(val_w, acc_sh.at[idx_w], sem2, add=True)
            return c
        lax.fori_loop(0, NBLK // 2, pair, 0)
        pltpu.make_async_copy(val_v, acc_sh.at[idx_v], sem).wait()
        pltpu.make_async_copy(val_w, acc_sh.at[idx_w], sem2).wait()
        plsc.subcore_barrier()

        pltpu.sync_copy(acc_sh.at[pl.ds(sid * ZSEG, ZSEG)], seg_v)
        pltpu.sync_copy(seg_v, p_hbm.at[pl.ds(cid * ACC + sid * ZSEG, ZSEG)])


def _combine(consts_ref, p0_ref, p1_ref, x3_ref, out_ref):
    wl = consts_ref[0]
    wr = consts_ref[1]
    w10 = consts_ref[2]
    w11 = consts_ref[3]
    b10 = consts_ref[4]
    b11 = consts_ref[5]
    w20 = consts_ref[6]
    w21 = consts_ref[7]
    b2v = consts_ref[8]
    h = wl * (p0_ref[...] + p1_ref[...]) + wr * x3_ref[...]
    a0 = jnp.maximum(w10 * h + b10, 0.0)
    a1 = jnp.maximum(w11 * h + b11, 0.0)
    out_ref[...] = w20 * a0 + w21 * a1 + b2v


def kernel(x, edge_index, W_l, W_r, w1, b1, w2, b2):
    xf = x.reshape(-1)
    x_p = jnp.concatenate(
        [xf, jnp.zeros((X_PAD - N_NODES,), jnp.float32)])
    src = edge_index[0]
    dst = edge_index[1]
    npad = E_PAD - N_EDGES
    src_p = jnp.concatenate([src, jnp.zeros((npad,), jnp.int32)])
    dst_p = jnp.concatenate([dst, jnp.ones((npad,), jnp.int32)])

    mesh = plsc.VectorSubcoreMesh(core_axis_name="c", subcore_axis_name="s")
    partials = pl.kernel(
        _sc_scatter,
        out_type=jax.ShapeDtypeStruct((NC * ACC,), jnp.float32),
        mesh=mesh,
        compiler_params=pltpu.CompilerParams(needs_layout_passes=False),
        scratch_types=[
            pltpu.VMEM((X_PAD,), jnp.float32),
            pltpu.VMEM((BLK,), jnp.int32),
            pltpu.VMEM((BLK,), jnp.int32),
            pltpu.VMEM((BLK,), jnp.int32),
            pltpu.VMEM((BLK,), jnp.float32),
            pltpu.VMEM((BLK,), jnp.int32),
            pltpu.VMEM((BLK,), jnp.float32),
            pltpu.VMEM((ZSEG,), jnp.float32),
            pltpu.VMEM_SHARED((ACC,), jnp.float32),
            pltpu.SemaphoreType.DMA,
            pltpu.SemaphoreType.DMA,
        ],
    )(x_p, src_p, dst_p)

    x3 = xf[: N_OUT * 3].reshape(N_OUT, 3)[:, 0]
    x3_p = jnp.concatenate(
        [x3, jnp.zeros((ACC - N_OUT,), jnp.float32)]).reshape(264, 128)
    consts = jnp.concatenate([
        W_l.reshape(-1), W_r.reshape(-1), w1.reshape(-1),
        b1.reshape(-1), w2.reshape(-1), b2.reshape(-1),
        jnp.zeros((7,), jnp.float32),
    ])

    out2d = pl.pallas_call(
        _combine,
        out_shape=jax.ShapeDtypeStruct((264, 128), jnp.float32),
        in_specs=[
            pl.BlockSpec(memory_space=pltpu.SMEM),
            pl.BlockSpec(memory_space=pltpu.VMEM),
            pl.BlockSpec(memory_space=pltpu.VMEM),
            pl.BlockSpec(memory_space=pltpu.VMEM),
        ],
        out_specs=pl.BlockSpec(memory_space=pltpu.VMEM),
    )(consts, partials[:ACC].reshape(264, 128),
      partials[ACC:].reshape(264, 128), x3_p)

    return out2d.reshape(-1)[:N_OUT]


# speedup vs baseline: 4.5144x; 1.8388x over previous
import jax
import jax.numpy as jnp
from jax import lax
from jax.experimental import pallas as pl
from jax.experimental.pallas import tpu as pltpu
from jax.experimental.pallas import tpu_sc as plsc

N_NODES = 99999
N_EDGES = N_NODES * 64
N_OUT = N_NODES // 3
ACC = 33792
X_PAD = 100000

NC, NS = 2, 16
NW = NC * NS
BLK = 4096
NBLK = 50
W_E = NBLK * BLK
E_PAD = NW * W_E
ZSEG = ACC // NS


def _sc_scatter(x_hbm, src_hbm, dst_hbm, p_hbm,
                x_v, src_v, dst_v, idx_v, val_v, idx_w, val_w,
                seg_v, acc_sh, sem, sem2):
    cid = lax.axis_index("c")
    sid = lax.axis_index("s")
    wid = sid * NC + cid

    if True:
        pltpu.sync_copy(x_hbm, x_v)

        def zstep(j, c):
            seg_v[pl.ds(j * 16, 16)] = jnp.zeros((16,), jnp.float32)
            return c
        lax.fori_loop(0, ZSEG // 16, zstep, 0)
        pltpu.sync_copy(seg_v, acc_sh.at[pl.ds(sid * ZSEG, ZSEG)])
        plsc.subcore_barrier()

        base = wid * W_E
        three = jnp.int32(3)
        third = jnp.float32(1.0 / 3.0)

        def fill(boff, idx_b, val_b):
            pltpu.sync_copy(src_hbm.at[pl.ds(boff, BLK)], src_v)
            pltpu.sync_copy(dst_hbm.at[pl.ds(boff, BLK)], dst_v)

            def group(t, c2):
                co = t * 128
                for j in range(8):
                    o = co + j * 16
                    s16 = src_v[pl.ds(o, 16)]
                    d16 = dst_v[pl.ds(o, 16)]
                    xs = plsc.load_gather(x_v, [s16])
                    q = (d16.astype(jnp.float32) * third).astype(jnp.int32)
                    r = d16 - q * three
                    idx_b[pl.ds(o, 16)] = q
                    val_b[pl.ds(o, 16)] = jnp.where(
                        r == 0, xs, jnp.float32(0.0))
                return c2
            lax.fori_loop(0, BLK // 128, group, 0)

        def pair(h, c):
            boff = base + h * (2 * BLK)

            @pl.when(h > 0)
            def _():
                pltpu.make_async_copy(
                    val_v, acc_sh.at[idx_v], sem).wait()
            fill(boff, idx_v, val_v)
            pltpu.async_copy(val_v, acc_sh.at[idx_v], sem, add=True)

            @pl.when(h > 0)
            def _():
                pltpu.make_async_copy(
                    val_w, acc_sh.at[idx_w], sem2).wait()
            fill(boff + BLK, idx_w, val_w)
            pltpu.async_copy(val_w, acc_sh.at[idx_w], sem2, add=True)
            return c
        lax.fori_loop(0, NBLK // 2, pair, 0)
        pltpu.make_async_copy(val_v, acc_sh.at[idx_v], sem).wait()
        pltpu.make_async_copy(val_w, acc_sh.at[idx_w], sem2).wait()
        plsc.subcore_barrier()

        pltpu.sync_copy(acc_sh.at[pl.ds(sid * ZSEG, ZSEG)], seg_v)
        pltpu.sync_copy(seg_v, p_hbm.at[pl.ds(cid * ACC + sid * ZSEG, ZSEG)])


def _combine(consts_ref, p0_ref, p1_ref, x3_ref, out_ref):
    wl = consts_ref[0]
    wr = consts_ref[1]
    w10 = consts_ref[2]
    w11 = consts_ref[3]
    b10 = consts_ref[4]
    b11 = consts_ref[5]
    w20 = consts_ref[6]
    w21 = consts_ref[7]
    b2v = consts_ref[8]
    h = wl * (p0_ref[...] + p1_ref[...]) + wr * x3_ref[...]
    a0 = jnp.maximum(w10 * h + b10, 0.0)
    a1 = jnp.maximum(w11 * h + b11, 0.0)
    out_ref[...] = w20 * a0 + w21 * a1 + b2v


def kernel(x, edge_index, W_l, W_r, w1, b1, w2, b2):
    xf = x.reshape(-1)
    x_p = jnp.concatenate(
        [xf, jnp.zeros((X_PAD - N_NODES,), jnp.float32)])
    src = edge_index[0]
    dst = edge_index[1]
    npad = E_PAD - N_EDGES
    src_p = jnp.concatenate([src, jnp.zeros((npad,), jnp.int32)])
    dst_p = jnp.concatenate([dst, jnp.ones((npad,), jnp.int32)])

    mesh = plsc.VectorSubcoreMesh(core_axis_name="c", subcore_axis_name="s")
    partials = pl.kernel(
        _sc_scatter,
        out_type=jax.ShapeDtypeStruct((NC * ACC,), jnp.float32),
        mesh=mesh,
        compiler_params=pltpu.CompilerParams(needs_layout_passes=False),
        scratch_types=[
            pltpu.VMEM((X_PAD,), jnp.float32),
            pltpu.VMEM((BLK,), jnp.int32),
            pltpu.VMEM((BLK,), jnp.int32),
            pltpu.VMEM((BLK,), jnp.int32),
            pltpu.VMEM((BLK,), jnp.float32),
            pltpu.VMEM((BLK,), jnp.int32),
            pltpu.VMEM((BLK,), jnp.float32),
            pltpu.VMEM((ZSEG,), jnp.float32),
            pltpu.VMEM_SHARED((ACC,), jnp.float32),
            pltpu.SemaphoreType.DMA,
            pltpu.SemaphoreType.DMA,
        ],
    )(x_p, src_p, dst_p)

    x3 = xf[: N_OUT * 3].reshape(N_OUT, 3)[:, 0]
    x3_p = jnp.concatenate(
        [x3, jnp.zeros((ACC - N_OUT,), jnp.float32)]).reshape(264, 128)
    consts = jnp.concatenate([
        W_l.reshape(-1), W_r.reshape(-1), w1.reshape(-1),
        b1.reshape(-1), w2.reshape(-1), b2.reshape(-1),
        jnp.zeros((7,), jnp.float32),
    ])

    out2d = pl.pallas_call(
        _combine,
        out_shape=jax.ShapeDtypeStruct((264, 128), jnp.float32),
        in_specs=[
            pl.BlockSpec(memory_space=pltpu.SMEM),
            pl.BlockSpec(memory_space=pltpu.VMEM),
            pl.BlockSpec(memory_space=pltpu.VMEM),
            pl.BlockSpec(memory_space=pltpu.VMEM),
        ],
        out_specs=pl.BlockSpec(memory_space=pltpu.VMEM),
    )(consts, partials[:ACC].reshape(264, 128),
      partials[ACC:].reshape(264, 128), x3_p)

    return out2d.reshape(-1)[:N_OUT]


# speedup vs baseline: 5.4933x; 1.2168x over previous
import jax
import jax.numpy as jnp
from jax import lax
from jax.experimental import pallas as pl
from jax.experimental.pallas import tpu as pltpu
from jax.experimental.pallas import tpu_sc as plsc

N_NODES = 99999
N_EDGES = N_NODES * 64
N_OUT = N_NODES // 3
ACC = 33792
X_PAD = 100000

NC, NS = 2, 16
NW = NC * NS
BLK = 4096
NBLK = 50
W_E = NBLK * BLK
E_PAD = NW * W_E
ZSEG = ACC // NS


def _sc_scatter(x_hbm, src_hbm, dst_hbm, p_hbm,
                x_v, src_v, dst_v, idx_v, val_v, idx_w, val_w,
                seg_v, acc_sh, sem, sem2):
    cid = lax.axis_index("c")
    sid = lax.axis_index("s")
    wid = sid * NC + cid

    if True:
        pltpu.sync_copy(x_hbm, x_v)

        def zstep(j, c):
            seg_v[pl.ds(j * 16, 16)] = jnp.zeros((16,), jnp.float32)
            return c
        lax.fori_loop(0, ZSEG // 16, zstep, 0)
        pltpu.sync_copy(seg_v, acc_sh.at[pl.ds(sid * ZSEG, ZSEG)])
        plsc.subcore_barrier()

        base = wid * W_E
        three = jnp.int32(3)
        third = jnp.float32(1.0 / 3.0)

        def fill(boff, idx_b, val_b):
            pltpu.sync_copy(src_hbm.at[pl.ds(boff, BLK)], src_v)
            pltpu.sync_copy(dst_hbm.at[pl.ds(boff, BLK)], dst_v)

            def group(t, c2):
                co = t * 128
                for j in range(8):
                    o = co + j * 16
                    s16 = src_v[pl.ds(o, 16)]
                    d16 = dst_v[pl.ds(o, 16)]
                    xs = plsc.load_gather(x_v, [s16])
                    q = (d16.astype(jnp.float32) * third).astype(jnp.int32)
                    r = d16 - q * three
                    idx_b[pl.ds(o, 16)] = q
                    val_b[pl.ds(o, 16)] = jnp.where(
                        r == 0, xs, jnp.float32(0.0))
                return c2
            lax.fori_loop(0, BLK // 128, group, 0)

        def pair(h, c):
            boff = base + h * (2 * BLK)

            @pl.when(h > 0)
            def _():
                pltpu.make_async_copy(
                    val_v, acc_sh.at[idx_v], sem).wait()
            fill(boff, idx_v, val_v)
            pltpu.async_copy(val_v, acc_sh.at[idx_v], sem, add=True)

            @pl.when(h > 0)
            def _():
                pltpu.make_async_copy(
                    val_w, acc_sh.at[idx_w], sem2).wait()
            fill(boff + BLK, idx_w, val_w)
            pltpu.async_copy(val_w, acc_sh.at[idx_w], sem2, add=True)
            return c
        lax.fori_loop(0, NBLK // 2, pair, 0)
        pltpu.make_async_copy(val_v, acc_sh.at[idx_v], sem).wait()
        pltpu.make_async_copy(val_w, acc_sh.at[idx_w], sem2).wait()
        plsc.subcore_barrier()

        pltpu.sync_copy(acc_sh.at[pl.ds(sid * ZSEG, ZSEG)], seg_v)
        pltpu.sync_copy(seg_v, p_hbm.at[pl.ds(cid * ACC + sid * ZSEG, ZSEG)])


def _combine(consts_ref, p0_ref, p1_ref, x3_ref, out_ref):
    wl = consts_ref[0]
    wr = consts_ref[1]
    w10 = consts_ref[2]
    w11 = consts_ref[3]
    b10 = consts_ref[4]
    b11 = consts_ref[5]
    w20 = consts_ref[6]
    w21 = consts_ref[7]
    b2v = consts_ref[8]
    h = wl * (p0_ref[...] + p1_ref[...]) + wr * x3_ref[...]
    a0 = jnp.maximum(w10 * h + b10, 0.0)
    a1 = jnp.maximum(w11 * h + b11, 0.0)
    out_ref[...] = w20 * a0 + w21 * a1 + b2v


def kernel(x, edge_index, W_l, W_r, w1, b1, w2, b2):
    xf = x.reshape(-1)
    x_p = jnp.concatenate(
        [xf, jnp.zeros((X_PAD - N_NODES,), jnp.float32)])
    src = edge_index[0]
    dst = edge_index[1]
    npad = E_PAD - N_EDGES
    src_p = jnp.concatenate([src, jnp.zeros((npad,), jnp.int32)])
    pad_d = (jnp.arange(npad, dtype=jnp.int32) % jnp.int32(N_OUT)) * 3 + 1
    dst_p = jnp.concatenate([dst, pad_d])

    mesh = plsc.VectorSubcoreMesh(core_axis_name="c", subcore_axis_name="s")
    partials = pl.kernel(
        _sc_scatter,
        out_type=jax.ShapeDtypeStruct((NC * ACC,), jnp.float32),
        mesh=mesh,
        compiler_params=pltpu.CompilerParams(needs_layout_passes=False),
        scratch_types=[
            pltpu.VMEM((X_PAD,), jnp.float32),
            pltpu.VMEM((BLK,), jnp.int32),
            pltpu.VMEM((BLK,), jnp.int32),
            pltpu.VMEM((BLK,), jnp.int32),
            pltpu.VMEM((BLK,), jnp.float32),
            pltpu.VMEM((BLK,), jnp.int32),
            pltpu.VMEM((BLK,), jnp.float32),
            pltpu.VMEM((ZSEG,), jnp.float32),
            pltpu.VMEM_SHARED((ACC,), jnp.float32),
            pltpu.SemaphoreType.DMA,
            pltpu.SemaphoreType.DMA,
        ],
    )(x_p, src_p, dst_p)

    x3 = xf[: N_OUT * 3].reshape(N_OUT, 3)[:, 0]
    x3_p = jnp.concatenate(
        [x3, jnp.zeros((ACC - N_OUT,), jnp.float32)]).reshape(264, 128)
    consts = jnp.concatenate([
        W_l.reshape(-1), W_r.reshape(-1), w1.reshape(-1),
        b1.reshape(-1), w2.reshape(-1), b2.reshape(-1),
        jnp.zeros((7,), jnp.float32),
    ])

    out2d = pl.pallas_call(
        _combine,
        out_shape=jax.ShapeDtypeStruct((264, 128), jnp.float32),
        in_specs=[
            pl.BlockSpec(memory_space=pltpu.SMEM),
            pl.BlockSpec(memory_space=pltpu.VMEM),
            pl.BlockSpec(memory_space=pltpu.VMEM),
            pl.BlockSpec(memory_space=pltpu.VMEM),
        ],
        out_specs=pl.BlockSpec(memory_space=pltpu.VMEM),
    )(consts, partials[:ACC].reshape(264, 128),
      partials[ACC:].reshape(264, 128), x3_p)

    return out2d.reshape(-1)[:N_OUT]
